# Initial kernel scaffold; baseline (speedup 1.0000x reference)
import jax
import jax.numpy as jnp
from jax.experimental import pallas as pl


def kernel(x, edge_index, edge_weight, W, b):
    raise NotImplementedError("write your pallas kernel here")


# speedup vs baseline: 4.1429x; 4.1429x over previous
import functools

import jax
import jax.numpy as jnp
from jax import lax
from jax.experimental import pallas as pl
from jax.experimental.pallas import tpu as pltpu
from jax.experimental.pallas import tpu_sc as plsc

N = 10000
E = 320000
D = 128
NC = 2
NS = 16
NW = NC * NS
EPW = E // NW
CH = 80
NCH = EPW // CH
RPS = 624
TAIL = N - NS * RPS
ZB = 78


def _spmm_body(x_hbm, src_hbm, dst_hbm, w_hbm, out_hbm,
               src_v, dst_v, w_v, rows, zbuf, h_sh, sem):
    c = lax.axis_index("c")
    s = lax.axis_index("s")
    gwid = c * NS + s

    zero16 = jnp.zeros((16,), jnp.float32)

    def _zrow(i, carry):
        for v in range(D // 16):
            zbuf[i, pl.ds(v * 16, 16)] = zero16
        return carry

    lax.fori_loop(0, ZB, _zrow, 0)
    row0 = s * RPS
    for k in range(RPS // ZB):
        pltpu.sync_copy(zbuf, h_sh.at[pl.ds(row0 + k * ZB, ZB)])

    @pl.when(s == NS - 1)
    def _zero_tail():
        pltpu.sync_copy(zbuf.at[pl.ds(0, TAIL)], h_sh.at[pl.ds(NS * RPS, TAIL)])

    plsc.subcore_barrier()

    def _chunk(j, carry):
        base = pl.multiple_of(gwid * EPW + j * CH, 8)
        pltpu.sync_copy(src_hbm.at[pl.ds(base, CH)], src_v)
        pltpu.sync_copy(dst_hbm.at[pl.ds(base, CH)], dst_v)
        pltpu.sync_copy(w_hbm.at[pl.ds(base, CH)], w_v)
        pltpu.async_copy(x_hbm.at[src_v], rows, sem).wait()
        for g in range(CH // 16):
            w16 = w_v[pl.ds(g * 16, 16)]
            for i in range(16):
                r = g * 16 + i
                wb = jnp.full((16,), w16[i], jnp.float32)
                for v in range(D // 16):
                    rows[r, pl.ds(v * 16, 16)] = rows[r, pl.ds(v * 16, 16)] * wb
        pltpu.sync_copy(rows, h_sh.at[dst_v], add=True)
        return carry

    lax.fori_loop(0, NCH, _chunk, 0)
    plsc.subcore_barrier()

    pltpu.sync_copy(h_sh.at[pl.ds(row0, RPS)], out_hbm.at[c, pl.ds(row0, RPS)])

    @pl.when(s == NS - 1)
    def _write_tail():
        pltpu.sync_copy(h_sh.at[pl.ds(NS * RPS, TAIL)],
                        out_hbm.at[c, pl.ds(NS * RPS, TAIL)])


@functools.partial(jax.jit, static_argnames=())
def _spmm(x, src, dst, w):
    mesh = plsc.VectorSubcoreMesh(core_axis_name="c", subcore_axis_name="s")
    f = pl.kernel(
        _spmm_body,
        out_type=jax.ShapeDtypeStruct((NC, N, D), jnp.float32),
        mesh=mesh,
        scratch_types=[
            pltpu.VMEM((CH,), jnp.int32),
            pltpu.VMEM((CH,), jnp.int32),
            pltpu.VMEM((CH,), jnp.float32),
            pltpu.VMEM((CH, D), jnp.float32),
            pltpu.VMEM((ZB, D), jnp.float32),
            pltpu.VMEM_SHARED((N, D), jnp.float32),
            pltpu.SemaphoreType.DMA,
        ],
    )
    return f(x, src, dst, w)


def _linear_body(p0_ref, p1_ref, w_ref, b_ref, o_ref):
    h = p0_ref[...] + p1_ref[...]
    acc = jnp.dot(h, w_ref[...], preferred_element_type=jnp.float32)
    o_ref[...] = jnp.maximum(acc + b_ref[...], 0.0)


def _linear(partials, W, b):
    blk = 1000
    grid = (N // blk,)
    return pl.pallas_call(
        _linear_body,
        grid=grid,
        in_specs=[
            pl.BlockSpec((blk, D), lambda i: (i, 0)),
            pl.BlockSpec((blk, D), lambda i: (i, 0)),
            pl.BlockSpec((D, D), lambda i: (0, 0)),
            pl.BlockSpec((1, D), lambda i: (0, 0)),
        ],
        out_specs=pl.BlockSpec((blk, D), lambda i: (i, 0)),
        out_shape=jax.ShapeDtypeStruct((N, D), jnp.float32),
    )(partials[0], partials[1], W, b.reshape(1, D))


def kernel(x, edge_index, edge_weight, W, b):
    src = edge_index[0]
    dst = edge_index[1]
    partials = _spmm(x, src, dst, edge_weight)
    return _linear(partials, W, b)


# speedup vs baseline: 4.4920x; 1.0843x over previous
import jax
import jax.numpy as jnp
from jax import lax
from jax.experimental import pallas as pl
from jax.experimental.pallas import tpu as pltpu
from jax.experimental.pallas import tpu_sc as plsc

N = 10000
E = 320000
D = 128
DH = D // 2
NC = 2
NS = 16
EPT = E // NS
CH = 80
G = 5
NG = EPT // (CH * G)
RPS = 624
TAIL = N - NS * RPS
ZB = 78


def _scale_rows(rows_b, w_blk, b):
    def wgroup(k, carry):
        w16 = w_blk[b, pl.ds(k * 16, 16)]
        for i in range(16):
            wb = jnp.full((16,), w16[i], jnp.float32)
            r = k * 16 + i
            for v in range(DH // 16):
                rows_b[r, pl.ds(v * 16, 16)] = rows_b[r, pl.ds(v * 16, 16)] * wb
        return carry

    lax.fori_loop(0, CH // 16, wgroup, 0)


def _spmm_body(x_hbm, src_hbm, dst_hbm, w_hbm, out_hbm,
               srcA, dstA, wA, srcB, dstB, wB, rows, zbuf, h_sh,
               gsems, ssems, isemA, isemB):
    c = lax.axis_index("c")
    s = lax.axis_index("s")

    def load_idx_block(g, bufs, sem):
        return [pltpu.async_copy(src_hbm.at[s, g], bufs[0], sem),
                pltpu.async_copy(dst_hbm.at[s, g], bufs[1], sem),
                pltpu.async_copy(w_hbm.at[s, g], bufs[2], sem)]

    def wait_idx_block(bufs, sem):
        for src_r, buf in zip((src_hbm.at[s, 0], dst_hbm.at[s, 0],
                               w_hbm.at[s, 0]), bufs):
            pltpu.make_async_copy(src_r, buf, sem).wait()

    def gather(src_blk, b):
        return pltpu.async_copy(x_hbm.at[c].at[src_blk.at[b]], rows.at[b],
                                gsems[b])

    def wait_gather(src_blk, b):
        pltpu.make_async_copy(x_hbm.at[c].at[src_blk.at[b]], rows.at[b],
                              gsems[b]).wait()

    def scatter(dst_blk, b):
        return pltpu.async_copy(rows.at[b], h_sh.at[dst_blk.at[b]], ssems[b],
                                add=True)

    def wait_scatter(dst_blk, b):
        pltpu.make_async_copy(rows.at[b], h_sh.at[dst_blk.at[b]],
                              ssems[b]).wait()

    for d in load_idx_block(0, (srcA, dstA, wA), isemA):
        d.wait()
    for b in range(G):
        gather(srcA, b)

    zero16 = jnp.zeros((16,), jnp.float32)

    def _zrow(i, carry):
        for v in range(DH // 16):
            zbuf[i, pl.ds(v * 16, 16)] = zero16
        return carry

    lax.fori_loop(0, ZB, _zrow, 0)
    row0 = s * RPS
    for k in range(RPS // ZB):
        pltpu.sync_copy(zbuf, h_sh.at[pl.ds(row0 + k * ZB, ZB)])

    @pl.when(s == NS - 1)
    def _zero_tail():
        pltpu.sync_copy(zbuf.at[pl.ds(0, TAIL)], h_sh.at[pl.ds(NS * RPS, TAIL)])

    plsc.subcore_barrier()

    def _pair(m, carry):
        g0 = 2 * m
        load_idx_block(g0 + 1, (srcB, dstB, wB), isemB)
        for b in range(G):
            wait_gather(srcA, b)
            _scale_rows(rows.at[b], wA, b)
            scatter(dstA, b)
        wait_idx_block((srcB, dstB, wB), isemB)
        for b in range(G):
            wait_scatter(dstA, b)
            gather(srcB, b)

        @pl.when(m < NG // 2 - 1)
        def _prefetch_a():
            load_idx_block(g0 + 2, (srcA, dstA, wA), isemA)

        for b in range(G):
            wait_gather(srcB, b)
            _scale_rows(rows.at[b], wB, b)
            scatter(dstB, b)

        @pl.when(m < NG // 2 - 1)
        def _next_gathers():
            wait_idx_block((srcA, dstA, wA), isemA)
            for b in range(G):
                wait_scatter(dstB, b)
                gather(srcA, b)

        return carry

    lax.fori_loop(0, NG // 2, _pair, 0)
    for b in range(G):
        wait_scatter(dstB, b)
    plsc.subcore_barrier()

    pltpu.sync_copy(h_sh.at[pl.ds(row0, RPS)],
                    out_hbm.at[c].at[pl.ds(row0, RPS)])

    @pl.when(s == NS - 1)
    def _write_tail():
        pltpu.sync_copy(h_sh.at[pl.ds(NS * RPS, TAIL)],
                        out_hbm.at[c].at[pl.ds(NS * RPS, TAIL)])


def _spmm(x2, src4, dst4, w4):
    mesh = plsc.VectorSubcoreMesh(core_axis_name="c", subcore_axis_name="s")

    def body(x_hbm, src_hbm, dst_hbm, w_hbm, out_hbm, srcA, dstA, wA,
             srcB, dstB, wB, rows, zbuf, h_sh,
             g0, g1, g2, g3, g4, s0, s1, s2, s3, s4, iA, iB):
        _spmm_body(x_hbm, src_hbm, dst_hbm, w_hbm, out_hbm, srcA, dstA, wA,
                   srcB, dstB, wB, rows, zbuf, h_sh,
                   [g0, g1, g2, g3, g4], [s0, s1, s2, s3, s4], iA, iB)

    f = pl.kernel(
        body,
        out_type=jax.ShapeDtypeStruct((NC, N, DH), jnp.float32),
        mesh=mesh,
        scratch_types=[
            pltpu.VMEM((G, CH), jnp.int32),
            pltpu.VMEM((G, CH), jnp.int32),
            pltpu.VMEM((G, CH), jnp.float32),
            pltpu.VMEM((G, CH), jnp.int32),
            pltpu.VMEM((G, CH), jnp.int32),
            pltpu.VMEM((G, CH), jnp.float32),
            pltpu.VMEM((G, CH, DH), jnp.float32),
            pltpu.VMEM((ZB, DH), jnp.float32),
            pltpu.VMEM_SHARED((N, DH), jnp.float32),
        ] + [pltpu.SemaphoreType.DMA] * 12,
        compiler_params=pltpu.CompilerParams(use_tc_tiling_on_sc=False),
    )
    return f(x2, src4, dst4, w4)


def _linear_body(h0_ref, h1_ref, w_ref, b_ref, o_ref):
    h = jnp.concatenate([h0_ref[0], h1_ref[0]], axis=1)
    acc = jnp.dot(h, w_ref[...], preferred_element_type=jnp.float32)
    o_ref[...] = jnp.maximum(acc + b_ref[...], 0.0)


def _linear(h2, W, b):
    blk = 1000
    grid = (N // blk,)
    return pl.pallas_call(
        _linear_body,
        grid=grid,
        in_specs=[
            pl.BlockSpec((1, blk, DH), lambda i: (0, i, 0)),
            pl.BlockSpec((1, blk, DH), lambda i: (1, i, 0)),
            pl.BlockSpec((D, D), lambda i: (0, 0)),
            pl.BlockSpec((1, D), lambda i: (0, 0)),
        ],
        out_specs=pl.BlockSpec((blk, D), lambda i: (i, 0)),
        out_shape=jax.ShapeDtypeStruct((N, D), jnp.float32),
    )(h2, h2, W, b.reshape(1, D))


def kernel(x, edge_index, edge_weight, W, b):
    x2 = jnp.stack([x[:, :DH], x[:, DH:]])
    src4 = edge_index[0].reshape(NS, NG, G, CH)
    dst4 = edge_index[1].reshape(NS, NG, G, CH)
    w4 = edge_weight.reshape(NS, NG, G, CH)
    h2 = _spmm(x2, src4, dst4, w4)
    return _linear(h2, W, b)


# speedup vs baseline: 5.1197x; 1.1397x over previous
import jax
import jax.numpy as jnp
from jax import lax
from jax.experimental import pallas as pl
from jax.experimental.pallas import tpu as pltpu
from jax.experimental.pallas import tpu_sc as plsc

N = 10000
E = 320000
D = 128
DH = D // 2
NC = 2
NS = 16
EPT = E // NS
CH = 80
G = 5
NG = EPT // (CH * G)
RPS = 624
TAIL = N - NS * RPS
ZB = 78


def _scale_rows(rows_b, w_blk, b):
    @plsc.parallel_loop(0, CH, unroll=8)
    def body(r):
        wb = w_blk[b, r]
        for v in range(DH // 16):
            rows_b[r, pl.ds(v * 16, 16)] = rows_b[r, pl.ds(v * 16, 16)] * wb


def _spmm_body(x_hbm, src_hbm, dst_hbm, w_hbm, out_hbm,
               srcA, dstA, wA, srcB, dstB, wB, rows, zbuf, h_sh,
               gsems, ssems, isemA, isemB):
    c = lax.axis_index("c")
    s = lax.axis_index("s")

    def load_idx_block(g, bufs, sem):
        return [pltpu.async_copy(src_hbm.at[s, g], bufs[0], sem),
                pltpu.async_copy(dst_hbm.at[s, g], bufs[1], sem),
                pltpu.async_copy(w_hbm.at[s, g], bufs[2], sem)]

    def wait_idx_block(bufs, sem):
        for src_r, buf in zip((src_hbm.at[s, 0], dst_hbm.at[s, 0],
                               w_hbm.at[s, 0]), bufs):
            pltpu.make_async_copy(src_r, buf, sem).wait()

    def gather(src_blk, b):
        return pltpu.async_copy(x_hbm.at[c].at[src_blk.at[b]], rows.at[b],
                                gsems[b])

    def wait_gather(src_blk, b):
        pltpu.make_async_copy(x_hbm.at[c].at[src_blk.at[b]], rows.at[b],
                              gsems[b]).wait()

    def scatter(dst_blk, b):
        return pltpu.async_copy(rows.at[b], h_sh.at[dst_blk.at[b]], ssems[b],
                                add=True)

    def wait_scatter(dst_blk, b):
        pltpu.make_async_copy(rows.at[b], h_sh.at[dst_blk.at[b]],
                              ssems[b]).wait()

    for d in load_idx_block(0, (srcA, dstA, wA), isemA):
        d.wait()
    for b in range(G):
        gather(srcA, b)

    zero16 = jnp.zeros((16,), jnp.float32)

    def _zrow(i, carry):
        for v in range(DH // 16):
            zbuf[i, pl.ds(v * 16, 16)] = zero16
        return carry

    lax.fori_loop(0, ZB, _zrow, 0)
    row0 = s * RPS
    for k in range(RPS // ZB):
        pltpu.sync_copy(zbuf, h_sh.at[pl.ds(row0 + k * ZB, ZB)])

    @pl.when(s == NS - 1)
    def _zero_tail():
        pltpu.sync_copy(zbuf.at[pl.ds(0, TAIL)], h_sh.at[pl.ds(NS * RPS, TAIL)])

    plsc.subcore_barrier()

    def _pair(m, carry):
        g0 = 2 * m
        load_idx_block(g0 + 1, (srcB, dstB, wB), isemB)
        for b in range(G):
            wait_gather(srcA, b)
            _scale_rows(rows.at[b], wA, b)
            scatter(dstA, b)
        wait_idx_block((srcB, dstB, wB), isemB)
        for b in range(G):
            wait_scatter(dstA, b)
            gather(srcB, b)

        @pl.when(m < NG // 2 - 1)
        def _prefetch_a():
            load_idx_block(g0 + 2, (srcA, dstA, wA), isemA)

        for b in range(G):
            wait_gather(srcB, b)
            _scale_rows(rows.at[b], wB, b)
            scatter(dstB, b)

        @pl.when(m < NG // 2 - 1)
        def _next_gathers():
            wait_idx_block((srcA, dstA, wA), isemA)
            for b in range(G):
                wait_scatter(dstB, b)
                gather(srcA, b)

        return carry

    lax.fori_loop(0, NG // 2, _pair, 0)
    for b in range(G):
        wait_scatter(dstB, b)
    plsc.subcore_barrier()

    pltpu.sync_copy(h_sh.at[pl.ds(row0, RPS)],
                    out_hbm.at[c].at[pl.ds(row0, RPS)])

    @pl.when(s == NS - 1)
    def _write_tail():
        pltpu.sync_copy(h_sh.at[pl.ds(NS * RPS, TAIL)],
                        out_hbm.at[c].at[pl.ds(NS * RPS, TAIL)])


def _spmm(x2, src4, dst4, w4):
    mesh = plsc.VectorSubcoreMesh(core_axis_name="c", subcore_axis_name="s")

    def body(x_hbm, src_hbm, dst_hbm, w_hbm, out_hbm, srcA, dstA, wA,
             srcB, dstB, wB, rows, zbuf, h_sh,
             g0, g1, g2, g3, g4, s0, s1, s2, s3, s4, iA, iB):
        _spmm_body(x_hbm, src_hbm, dst_hbm, w_hbm, out_hbm, srcA, dstA, wA,
                   srcB, dstB, wB, rows, zbuf, h_sh,
                   [g0, g1, g2, g3, g4], [s0, s1, s2, s3, s4], iA, iB)

    f = pl.kernel(
        body,
        out_type=jax.ShapeDtypeStruct((NC, N, DH), jnp.float32),
        mesh=mesh,
        scratch_types=[
            pltpu.VMEM((G, CH), jnp.int32),
            pltpu.VMEM((G, CH), jnp.int32),
            pltpu.VMEM((G, CH, 16), jnp.float32),
            pltpu.VMEM((G, CH), jnp.int32),
            pltpu.VMEM((G, CH), jnp.int32),
            pltpu.VMEM((G, CH, 16), jnp.float32),
            pltpu.VMEM((G, CH, DH), jnp.float32),
            pltpu.VMEM((ZB, DH), jnp.float32),
            pltpu.VMEM_SHARED((N, DH), jnp.float32),
        ] + [pltpu.SemaphoreType.DMA] * 12,
        compiler_params=pltpu.CompilerParams(use_tc_tiling_on_sc=False),
    )
    return f(x2, src4, dst4, w4)


def _linear_body(h0_ref, h1_ref, w_ref, b_ref, o_ref):
    h = jnp.concatenate([h0_ref[0], h1_ref[0]], axis=1)
    acc = jnp.dot(h, w_ref[...], preferred_element_type=jnp.float32)
    o_ref[...] = jnp.maximum(acc + b_ref[...], 0.0)


def _linear(h2, W, b):
    blk = 1000
    grid = (N // blk,)
    return pl.pallas_call(
        _linear_body,
        grid=grid,
        in_specs=[
            pl.BlockSpec((1, blk, DH), lambda i: (0, i, 0)),
            pl.BlockSpec((1, blk, DH), lambda i: (1, i, 0)),
            pl.BlockSpec((D, D), lambda i: (0, 0)),
            pl.BlockSpec((1, D), lambda i: (0, 0)),
        ],
        out_specs=pl.BlockSpec((blk, D), lambda i: (i, 0)),
        out_shape=jax.ShapeDtypeStruct((N, D), jnp.float32),
    )(h2, h2, W, b.reshape(1, D))


def kernel(x, edge_index, edge_weight, W, b):
    x2 = jnp.stack([x[:, :DH], x[:, DH:]])
    src4 = edge_index[0].reshape(NS, NG, G, CH)
    dst4 = edge_index[1].reshape(NS, NG, G, CH)
    w4 = jnp.broadcast_to(edge_weight[:, None], (E, 16)).reshape(NS, NG, G, CH, 16)
    h2 = _spmm(x2, src4, dst4, w4)
    return _linear(h2, W, b)
